# baseline (speedup 1.0000x reference)
import jax
import jax.numpy as jnp
from jax import lax
from jax.experimental import pallas as pl
from jax.experimental.pallas import tpu as pltpu
from jax.experimental.pallas import tpu_sc as plsc

_GAMMA = 12.0
_D = 128
_B = 16384
_NC = 2
_NS = 16
_NW = _NC * _NS
_PT = _B // _NW
_CH = 128
_NCH = _PT // _CH
_L = 16
_VPS = _D // _L


def _body(sample_hbm, tab_hbm, out_hbm,
          samp_v, h_v, r_v, t_v, hbuf, rbuf, tbuf, ubuf, out_v, sems):
    c = lax.axis_index("c")
    s = lax.axis_index("s")
    wid = s * _NC + c
    base = wid * _PT

    pltpu.sync_copy(sample_hbm.at[:, pl.ds(base, _PT)], samp_v)
    iota = lax.iota(jnp.int32, _L)
    zero = jnp.zeros((_L,), jnp.int32)

    def split(v, carry):
        cols = iota + v * _L
        h_v[pl.ds(v * _L, _L)] = plsc.load_gather(samp_v, [zero, cols])
        r_v[pl.ds(v * _L, _L)] = plsc.load_gather(samp_v, [zero + 1, cols]) + 1000
        t_v[pl.ds(v * _L, _L)] = plsc.load_gather(samp_v, [zero + 2, cols])
        return carry

    lax.fori_loop(0, _PT // _L, split, 0)

    def fire(k, slot):
        ks = pl.ds(k * _CH, _CH)
        pltpu.async_copy(tab_hbm.at[h_v.at[ks]], hbuf.at[slot], sems.at[slot, 0])
        pltpu.async_copy(tab_hbm.at[r_v.at[ks]], rbuf.at[slot], sems.at[slot, 1])
        pltpu.async_copy(tab_hbm.at[t_v.at[ks]], tbuf.at[slot], sems.at[slot, 2])

    fire(0, 0)

    def chunk_body(k, carry):
        slot = lax.rem(k, 2)

        @pl.when(k + 1 < _NCH)
        def _():
            fire(k + 1, 1 - slot)

        dummy = tab_hbm.at[pl.ds(0, _CH)]
        pltpu.make_async_copy(dummy, hbuf.at[slot], sems.at[slot, 0]).wait()
        pltpu.make_async_copy(dummy, rbuf.at[slot], sems.at[slot, 1]).wait()
        pltpu.make_async_copy(dummy, tbuf.at[slot], sems.at[slot, 2]).wait()

        def sample_body(j, carry2):
            terms = []
            for v in range(_D // 32):
                e2 = plsc.bitcast(hbuf[slot, j, pl.ds(v * _L, _L)], jnp.bfloat16)
                r2 = plsc.bitcast(rbuf[slot, j, pl.ds(v * _L, _L)], jnp.bfloat16)
                t2 = plsc.bitcast(tbuf[slot, j, pl.ds(v * _L, _L)], jnp.bfloat16)
                ea, eb = plsc.unpack(e2, format=plsc.PackFormat.INTERLEAVED)
                ra, rb = plsc.unpack(r2, format=plsc.PackFormat.INTERLEAVED)
                ta, tb = plsc.unpack(t2, format=plsc.PackFormat.INTERLEAVED)
                terms.append(jnp.abs(ea + ra - ta))
                terms.append(jnp.abs(eb + rb - tb))
            while len(terms) > 1:
                terms = [terms[i] + terms[i + 1]
                         for i in range(0, len(terms), 2)]
            ubuf[pl.ds(j * 17, _L)] = terms[0]
            return carry2

        lax.fori_loop(0, _CH, sample_body, 0, unroll=2)

        iota17 = lax.iota(jnp.int32, _L) * 17

        def red_body(gi, carry2):
            cols = [plsc.load_gather(ubuf, [iota17 + (gi * _L * 17 + j)])
                    for j in range(_L)]
            while len(cols) > 1:
                cols = [cols[i] + cols[i + 1] for i in range(0, len(cols), 2)]
            out_v[pl.ds(k * _CH + gi * _L, _L)] = jnp.float32(_GAMMA) - cols[0]
            return carry2

        lax.fori_loop(0, _CH // _L, red_body, 0)
        return carry

    lax.fori_loop(0, _NCH, chunk_body, 0)

    pltpu.sync_copy(out_v, out_hbm.at[pl.ds(base, _PT)])


def kernel(sample, entity_embedding, relation_embedding):
    mesh = plsc.VectorSubcoreMesh(core_axis_name="c", subcore_axis_name="s")
    call = pl.kernel(
        _body,
        out_type=jax.ShapeDtypeStruct((_B,), jnp.float32),
        mesh=mesh,
        compiler_params=pltpu.CompilerParams(
            needs_layout_passes=False, use_tc_tiling_on_sc=False),
        scratch_types=[
            pltpu.VMEM((3, _PT), jnp.int32),
            pltpu.VMEM((_PT,), jnp.int32),
            pltpu.VMEM((_PT,), jnp.int32),
            pltpu.VMEM((_PT,), jnp.int32),
            pltpu.VMEM((2, _CH, _D // 2), jnp.int32),
            pltpu.VMEM((2, _CH, _D // 2), jnp.int32),
            pltpu.VMEM((2, _CH, _D // 2), jnp.int32),
            pltpu.VMEM((_CH * 17,), jnp.float32),
            pltpu.VMEM((_PT,), jnp.float32),
            pltpu.SemaphoreType.DMA((2, 3)),
        ],
    )
    tab = jnp.concatenate([entity_embedding, relation_embedding], axis=0)
    bf = tab.astype(jnp.bfloat16).reshape(tab.shape[0], -1, 2)
    packed = lax.bitcast_convert_type(bf, jnp.int32)
    score = call(sample.T, packed)
    return score.reshape(_B, 1)


# speedup vs baseline: 1.0377x; 1.0377x over previous
import jax
import jax.numpy as jnp
from jax import lax
from jax.experimental import pallas as pl
from jax.experimental.pallas import tpu as pltpu
from jax.experimental.pallas import tpu_sc as plsc

_GAMMA = 12.0
_D = 128
_B = 16384
_NC = 2
_NS = 16
_NW = _NC * _NS
_PT = _B // _NW
_CH = 64
_NCH = _PT // _CH
_L = 16
_VPS = _D // _L


def _body(sample_hbm, tab_hbm, out_hbm,
          samp_v, h_v, r_v, t_v, hbuf, rbuf, tbuf, ubuf, out_v, sems):
    c = lax.axis_index("c")
    s = lax.axis_index("s")
    wid = s * _NC + c
    base = wid * _PT

    pltpu.sync_copy(sample_hbm.at[:, pl.ds(base, _PT)], samp_v)
    iota = lax.iota(jnp.int32, _L)
    zero = jnp.zeros((_L,), jnp.int32)

    def split(v, carry):
        cols = iota + v * _L
        h_v[pl.ds(v * _L, _L)] = plsc.load_gather(samp_v, [zero, cols])
        r_v[pl.ds(v * _L, _L)] = plsc.load_gather(samp_v, [zero + 1, cols]) + 1000
        t_v[pl.ds(v * _L, _L)] = plsc.load_gather(samp_v, [zero + 2, cols])
        return carry

    lax.fori_loop(0, _PT // _L, split, 0)

    def fire(k, slot):
        ks = pl.ds(k * _CH, _CH)
        pltpu.async_copy(tab_hbm.at[h_v.at[ks]], hbuf.at[slot], sems.at[slot, 0])
        pltpu.async_copy(tab_hbm.at[r_v.at[ks]], rbuf.at[slot], sems.at[slot, 1])
        pltpu.async_copy(tab_hbm.at[t_v.at[ks]], tbuf.at[slot], sems.at[slot, 2])

    fire(0, 0)

    def chunk_body(k, carry):
        slot = lax.rem(k, 2)

        @pl.when(k + 1 < _NCH)
        def _():
            fire(k + 1, 1 - slot)

        dummy = tab_hbm.at[pl.ds(0, _CH)]
        pltpu.make_async_copy(dummy, hbuf.at[slot], sems.at[slot, 0]).wait()
        pltpu.make_async_copy(dummy, rbuf.at[slot], sems.at[slot, 1]).wait()
        pltpu.make_async_copy(dummy, tbuf.at[slot], sems.at[slot, 2]).wait()

        def sample_body(j, carry2):
            terms = []
            for v in range(_D // 32):
                e2 = plsc.bitcast(hbuf[slot, j, pl.ds(v * _L, _L)], jnp.bfloat16)
                r2 = plsc.bitcast(rbuf[slot, j, pl.ds(v * _L, _L)], jnp.bfloat16)
                t2 = plsc.bitcast(tbuf[slot, j, pl.ds(v * _L, _L)], jnp.bfloat16)
                ea, eb = plsc.unpack(e2, format=plsc.PackFormat.INTERLEAVED)
                ra, rb = plsc.unpack(r2, format=plsc.PackFormat.INTERLEAVED)
                ta, tb = plsc.unpack(t2, format=plsc.PackFormat.INTERLEAVED)
                terms.append(jnp.abs(ea + ra - ta))
                terms.append(jnp.abs(eb + rb - tb))
            while len(terms) > 1:
                terms = [terms[i] + terms[i + 1]
                         for i in range(0, len(terms), 2)]
            ubuf[pl.ds(j * 17, _L)] = terms[0]
            return carry2

        lax.fori_loop(0, _CH, sample_body, 0, unroll=2)

        iota17 = lax.iota(jnp.int32, _L) * 17

        def red_body(gi, carry2):
            cols = [plsc.load_gather(ubuf, [iota17 + (gi * _L * 17 + j)])
                    for j in range(_L)]
            while len(cols) > 1:
                cols = [cols[i] + cols[i + 1] for i in range(0, len(cols), 2)]
            out_v[pl.ds(k * _CH + gi * _L, _L)] = jnp.float32(_GAMMA) - cols[0]
            return carry2

        lax.fori_loop(0, _CH // _L, red_body, 0)
        return carry

    lax.fori_loop(0, _NCH, chunk_body, 0)

    pltpu.sync_copy(out_v, out_hbm.at[pl.ds(base, _PT)])


def kernel(sample, entity_embedding, relation_embedding):
    mesh = plsc.VectorSubcoreMesh(core_axis_name="c", subcore_axis_name="s")
    call = pl.kernel(
        _body,
        out_type=jax.ShapeDtypeStruct((_B,), jnp.float32),
        mesh=mesh,
        compiler_params=pltpu.CompilerParams(
            needs_layout_passes=False, use_tc_tiling_on_sc=False),
        scratch_types=[
            pltpu.VMEM((3, _PT), jnp.int32),
            pltpu.VMEM((_PT,), jnp.int32),
            pltpu.VMEM((_PT,), jnp.int32),
            pltpu.VMEM((_PT,), jnp.int32),
            pltpu.VMEM((2, _CH, _D // 2), jnp.int32),
            pltpu.VMEM((2, _CH, _D // 2), jnp.int32),
            pltpu.VMEM((2, _CH, _D // 2), jnp.int32),
            pltpu.VMEM((_CH * 17,), jnp.float32),
            pltpu.VMEM((_PT,), jnp.float32),
            pltpu.SemaphoreType.DMA((2, 3)),
        ],
    )
    tab = jnp.concatenate([entity_embedding, relation_embedding], axis=0)
    bf = tab.astype(jnp.bfloat16).reshape(tab.shape[0], -1, 2)
    packed = lax.bitcast_convert_type(bf, jnp.int32)
    score = call(sample.T, packed)
    return score.reshape(_B, 1)


# speedup vs baseline: 1.0399x; 1.0021x over previous
import jax
import jax.numpy as jnp
from jax import lax
from jax.experimental import pallas as pl
from jax.experimental.pallas import tpu as pltpu
from jax.experimental.pallas import tpu_sc as plsc

_GAMMA = 12.0
_NE = 1000
_D = 128
_B = 16384
_NC = 2
_NS = 16
_NW = _NC * _NS
_PT = _B // _NW
_CH = 64
_NCH = _PT // _CH
_L = 16
_VPS = _D // _L


def _body(sample_hbm, tab_hbm, out_hbm,
          samp_v, h_v, r_v, t_v, hbuf, rbuf, tbuf, ubuf, out_v, tmp_v,
          tab_sh, sems):
    c = lax.axis_index("c")
    s = lax.axis_index("s")
    wid = s * _NC + c
    base = wid * _PT

    rbase = s * (2 * _NE // _NS)
    pltpu.sync_copy(tab_hbm.at[pl.ds(rbase, 2 * _NE // _NS), :], tmp_v)
    pltpu.sync_copy(tmp_v, tab_sh.at[pl.ds(rbase, 2 * _NE // _NS), :])

    pltpu.sync_copy(sample_hbm.at[:, pl.ds(base, _PT)], samp_v)
    iota = lax.iota(jnp.int32, _L)
    zero = jnp.zeros((_L,), jnp.int32)

    def split(v, carry):
        cols = iota + v * _L
        h_v[pl.ds(v * _L, _L)] = plsc.load_gather(samp_v, [zero, cols])
        r_v[pl.ds(v * _L, _L)] = plsc.load_gather(samp_v, [zero + 1, cols]) + _NE
        t_v[pl.ds(v * _L, _L)] = plsc.load_gather(samp_v, [zero + 2, cols])
        return carry

    lax.fori_loop(0, _PT // _L, split, 0)

    def fire(k, slot):
        ks = pl.ds(k * _CH, _CH)
        pltpu.async_copy(tab_sh.at[h_v.at[ks]], hbuf.at[slot], sems.at[slot, 0])
        pltpu.async_copy(tab_sh.at[r_v.at[ks]], rbuf.at[slot], sems.at[slot, 1])
        pltpu.async_copy(tab_sh.at[t_v.at[ks]], tbuf.at[slot], sems.at[slot, 2])

    plsc.subcore_barrier()
    fire(0, 0)

    def chunk_body(k, carry):
        slot = lax.rem(k, 2)

        @pl.when(k + 1 < _NCH)
        def _():
            fire(k + 1, 1 - slot)

        dummy = tab_hbm.at[pl.ds(0, _CH)]
        pltpu.make_async_copy(dummy, hbuf.at[slot], sems.at[slot, 0]).wait()
        pltpu.make_async_copy(dummy, rbuf.at[slot], sems.at[slot, 1]).wait()
        pltpu.make_async_copy(dummy, tbuf.at[slot], sems.at[slot, 2]).wait()

        def sample_body(j, carry2):
            terms = []
            for v in range(_D // 32):
                e2 = plsc.bitcast(hbuf[slot, j, pl.ds(v * _L, _L)], jnp.bfloat16)
                r2 = plsc.bitcast(rbuf[slot, j, pl.ds(v * _L, _L)], jnp.bfloat16)
                t2 = plsc.bitcast(tbuf[slot, j, pl.ds(v * _L, _L)], jnp.bfloat16)
                ea, eb = plsc.unpack(e2, format=plsc.PackFormat.INTERLEAVED)
                ra, rb = plsc.unpack(r2, format=plsc.PackFormat.INTERLEAVED)
                ta, tb = plsc.unpack(t2, format=plsc.PackFormat.INTERLEAVED)
                terms.append(jnp.abs(ea + ra - ta))
                terms.append(jnp.abs(eb + rb - tb))
            while len(terms) > 1:
                terms = [terms[i] + terms[i + 1]
                         for i in range(0, len(terms), 2)]
            ubuf[pl.ds(j * 17, _L)] = terms[0]
            return carry2

        lax.fori_loop(0, _CH, sample_body, 0, unroll=2)

        iota17 = lax.iota(jnp.int32, _L) * 17

        def red_body(gi, carry2):
            cols = [plsc.load_gather(ubuf, [iota17 + (gi * _L * 17 + j)])
                    for j in range(_L)]
            while len(cols) > 1:
                cols = [cols[i] + cols[i + 1] for i in range(0, len(cols), 2)]
            out_v[pl.ds(k * _CH + gi * _L, _L)] = jnp.float32(_GAMMA) - cols[0]
            return carry2

        lax.fori_loop(0, _CH // _L, red_body, 0)
        return carry

    lax.fori_loop(0, _NCH, chunk_body, 0)

    pltpu.sync_copy(out_v, out_hbm.at[pl.ds(base, _PT)])


def kernel(sample, entity_embedding, relation_embedding):
    mesh = plsc.VectorSubcoreMesh(core_axis_name="c", subcore_axis_name="s")
    call = pl.kernel(
        _body,
        out_type=jax.ShapeDtypeStruct((_B,), jnp.float32),
        mesh=mesh,
        compiler_params=pltpu.CompilerParams(
            needs_layout_passes=False, use_tc_tiling_on_sc=False),
        scratch_types=[
            pltpu.VMEM((3, _PT), jnp.int32),
            pltpu.VMEM((_PT,), jnp.int32),
            pltpu.VMEM((_PT,), jnp.int32),
            pltpu.VMEM((_PT,), jnp.int32),
            pltpu.VMEM((2, _CH, _D // 2), jnp.int32),
            pltpu.VMEM((2, _CH, _D // 2), jnp.int32),
            pltpu.VMEM((2, _CH, _D // 2), jnp.int32),
            pltpu.VMEM((_CH * 17,), jnp.float32),
            pltpu.VMEM((_PT,), jnp.float32),
            pltpu.VMEM((2 * _NE // _NS, _D // 2), jnp.int32),
            pltpu.VMEM_SHARED((2 * _NE, _D // 2), jnp.int32),
            pltpu.SemaphoreType.DMA((2, 3)),
        ],
    )
    tab = jnp.concatenate([entity_embedding, relation_embedding], axis=0)
    bf = tab.astype(jnp.bfloat16).reshape(tab.shape[0], -1, 2)
    packed = lax.bitcast_convert_type(bf, jnp.int32)
    score = call(sample.T, packed)
    return score.reshape(_B, 1)


# speedup vs baseline: 1.0656x; 1.0247x over previous
import jax
import jax.numpy as jnp
from jax import lax
from jax.experimental import pallas as pl
from jax.experimental.pallas import tpu as pltpu
from jax.experimental.pallas import tpu_sc as plsc

_GAMMA = 12.0
_NE = 1000
_D = 128
_B = 16384
_NC = 2
_NS = 16
_NW = _NC * _NS
_PT = _B // _NW
_CH = 64
_NCH = _PT // _CH
_L = 16
_VPS = _D // _L


_RT = 2 * _NE // _NS


def _body(sample_hbm, ent_hbm, rel_hbm, out_hbm,
          samp_v, h_v, r_v, t_v, hbuf, rbuf, tbuf, ubuf, out_v, tmp_v,
          pk_v, tab_sh, sems):
    c = lax.axis_index("c")
    s = lax.axis_index("s")
    wid = s * _NC + c
    base = wid * _PT

    rbase = s * _RT

    @pl.when(s < _NS // 2)
    def _():
        pltpu.sync_copy(ent_hbm.at[pl.ds(rbase, _RT), :], tmp_v)

    @pl.when(s >= _NS // 2)
    def _():
        pltpu.sync_copy(rel_hbm.at[pl.ds(rbase - _NE, _RT), :], tmp_v)

    iota2 = lax.iota(jnp.int32, _L) * 2

    def pack_row(rr, carry):
        rsplat = jnp.zeros((_L,), jnp.int32) + rr
        for d0 in range(0, _D, 2 * _L):
            a = plsc.load_gather(tmp_v, [rsplat, iota2 + d0])
            b = plsc.load_gather(tmp_v, [rsplat, iota2 + (d0 + 1)])
            w = plsc.pack(a, b, format=plsc.PackFormat.INTERLEAVED)
            pk_v[rr, pl.ds(d0 // 2, _L)] = plsc.bitcast(w, jnp.int32)
        return carry

    lax.fori_loop(0, _RT, pack_row, 0, unroll=2)
    pltpu.sync_copy(pk_v, tab_sh.at[pl.ds(rbase, _RT), :])

    pltpu.sync_copy(sample_hbm.at[:, pl.ds(base, _PT)], samp_v)
    iota = lax.iota(jnp.int32, _L)
    zero = jnp.zeros((_L,), jnp.int32)

    def split(v, carry):
        cols = iota + v * _L
        h_v[pl.ds(v * _L, _L)] = plsc.load_gather(samp_v, [zero, cols])
        r_v[pl.ds(v * _L, _L)] = plsc.load_gather(samp_v, [zero + 1, cols]) + _NE
        t_v[pl.ds(v * _L, _L)] = plsc.load_gather(samp_v, [zero + 2, cols])
        return carry

    lax.fori_loop(0, _PT // _L, split, 0)

    def fire(k, slot):
        ks = pl.ds(k * _CH, _CH)
        pltpu.async_copy(tab_sh.at[h_v.at[ks]], hbuf.at[slot], sems.at[slot, 0])
        pltpu.async_copy(tab_sh.at[r_v.at[ks]], rbuf.at[slot], sems.at[slot, 1])
        pltpu.async_copy(tab_sh.at[t_v.at[ks]], tbuf.at[slot], sems.at[slot, 2])

    plsc.subcore_barrier()
    fire(0, 0)

    def chunk_body(k, carry):
        slot = lax.rem(k, 2)

        @pl.when(k + 1 < _NCH)
        def _():
            fire(k + 1, 1 - slot)

        dummy = tab_sh.at[pl.ds(0, _CH)]
        pltpu.make_async_copy(dummy, hbuf.at[slot], sems.at[slot, 0]).wait()
        pltpu.make_async_copy(dummy, rbuf.at[slot], sems.at[slot, 1]).wait()
        pltpu.make_async_copy(dummy, tbuf.at[slot], sems.at[slot, 2]).wait()

        def sample_body(j, carry2):
            terms = []
            for v in range(_D // 32):
                e2 = plsc.bitcast(hbuf[slot, j, pl.ds(v * _L, _L)], jnp.bfloat16)
                r2 = plsc.bitcast(rbuf[slot, j, pl.ds(v * _L, _L)], jnp.bfloat16)
                t2 = plsc.bitcast(tbuf[slot, j, pl.ds(v * _L, _L)], jnp.bfloat16)
                ea, eb = plsc.unpack(e2, format=plsc.PackFormat.INTERLEAVED)
                ra, rb = plsc.unpack(r2, format=plsc.PackFormat.INTERLEAVED)
                ta, tb = plsc.unpack(t2, format=plsc.PackFormat.INTERLEAVED)
                terms.append(jnp.abs(ea + ra - ta))
                terms.append(jnp.abs(eb + rb - tb))
            while len(terms) > 1:
                terms = [terms[i] + terms[i + 1]
                         for i in range(0, len(terms), 2)]
            ubuf[pl.ds(j * 17, _L)] = terms[0]
            return carry2

        lax.fori_loop(0, _CH, sample_body, 0, unroll=2)

        iota17 = lax.iota(jnp.int32, _L) * 17

        def red_body(gi, carry2):
            cols = [plsc.load_gather(ubuf, [iota17 + (gi * _L * 17 + j)])
                    for j in range(_L)]
            while len(cols) > 1:
                cols = [cols[i] + cols[i + 1] for i in range(0, len(cols), 2)]
            out_v[pl.ds(k * _CH + gi * _L, _L)] = jnp.float32(_GAMMA) - cols[0]
            return carry2

        lax.fori_loop(0, _CH // _L, red_body, 0)
        return carry

    lax.fori_loop(0, _NCH, chunk_body, 0)

    pltpu.sync_copy(out_v, out_hbm.at[pl.ds(base, _PT)])


def kernel(sample, entity_embedding, relation_embedding):
    mesh = plsc.VectorSubcoreMesh(core_axis_name="c", subcore_axis_name="s")
    call = pl.kernel(
        _body,
        out_type=jax.ShapeDtypeStruct((_B,), jnp.float32),
        mesh=mesh,
        compiler_params=pltpu.CompilerParams(
            needs_layout_passes=False, use_tc_tiling_on_sc=False),
        scratch_types=[
            pltpu.VMEM((3, _PT), jnp.int32),
            pltpu.VMEM((_PT,), jnp.int32),
            pltpu.VMEM((_PT,), jnp.int32),
            pltpu.VMEM((_PT,), jnp.int32),
            pltpu.VMEM((2, _CH, _D // 2), jnp.int32),
            pltpu.VMEM((2, _CH, _D // 2), jnp.int32),
            pltpu.VMEM((2, _CH, _D // 2), jnp.int32),
            pltpu.VMEM((_CH * 17,), jnp.float32),
            pltpu.VMEM((_PT,), jnp.float32),
            pltpu.VMEM((_RT, _D), jnp.float32),
            pltpu.VMEM((_RT, _D // 2), jnp.int32),
            pltpu.VMEM_SHARED((2 * _NE, _D // 2), jnp.int32),
            pltpu.SemaphoreType.DMA((2, 3)),
        ],
    )
    score = call(sample.T, entity_embedding, relation_embedding)
    return score.reshape(_B, 1)


# speedup vs baseline: 1.1139x; 1.0454x over previous
import jax
import jax.numpy as jnp
from jax import lax
from jax.experimental import pallas as pl
from jax.experimental.pallas import tpu as pltpu
from jax.experimental.pallas import tpu_sc as plsc

_GAMMA = 12.0
_NE = 1000
_D = 128
_B = 16384
_NC = 2
_NS = 16
_NW = _NC * _NS
_PT = _B // _NW
_CH = 64
_NCH = _PT // _CH
_L = 16
_VPS = _D // _L


_RT = 2 * _NE // _NS


def _body(sample_hbm, ent_hbm, rel_hbm, out_hbm,
          samp_v, h_v, r_v, t_v, hbuf, rbuf, tbuf, ubuf, out_v, tmp_v,
          pk_v, tab_sh, sems):
    c = lax.axis_index("c")
    s = lax.axis_index("s")
    wid = s * _NC + c
    base = wid * _PT

    rbase = s * _RT

    @pl.when(s < _NS // 2)
    def _():
        pltpu.sync_copy(ent_hbm.at[pl.ds(rbase, _RT), :], tmp_v)

    @pl.when(s >= _NS // 2)
    def _():
        pltpu.sync_copy(rel_hbm.at[pl.ds(rbase - _NE, _RT), :], tmp_v)

    iota2 = lax.iota(jnp.int32, _L) * 2

    def pack_row(rr, carry):
        rsplat = jnp.zeros((_L,), jnp.int32) + rr
        for d0 in range(0, _D, 2 * _L):
            a = plsc.load_gather(tmp_v, [rsplat, iota2 + d0])
            b = plsc.load_gather(tmp_v, [rsplat, iota2 + (d0 + 1)])
            w = plsc.pack(a, b, format=plsc.PackFormat.INTERLEAVED)
            pk_v[rr, pl.ds(d0 // 2, _L)] = plsc.bitcast(w, jnp.int32)
        return carry

    lax.fori_loop(0, _RT, pack_row, 0, unroll=2)
    pltpu.sync_copy(pk_v, tab_sh.at[pl.ds(rbase, _RT), :])

    pltpu.sync_copy(sample_hbm.at[:, pl.ds(base, _PT)], samp_v)
    iota = lax.iota(jnp.int32, _L)
    zero = jnp.zeros((_L,), jnp.int32)

    def split(v, carry):
        cols = iota + v * _L
        h_v[pl.ds(v * _L, _L)] = plsc.load_gather(samp_v, [zero, cols])
        r_v[pl.ds(v * _L, _L)] = plsc.load_gather(samp_v, [zero + 1, cols]) + _NE
        t_v[pl.ds(v * _L, _L)] = plsc.load_gather(samp_v, [zero + 2, cols])
        return carry

    lax.fori_loop(0, _PT // _L, split, 0)

    def fire(k, slot):
        ks = pl.ds(k * _CH, _CH)
        pltpu.async_copy(tab_sh.at[h_v.at[ks]], hbuf.at[slot], sems.at[slot, 0])
        pltpu.async_copy(tab_sh.at[r_v.at[ks]], rbuf.at[slot], sems.at[slot, 1])
        pltpu.async_copy(tab_sh.at[t_v.at[ks]], tbuf.at[slot], sems.at[slot, 2])

    plsc.subcore_barrier()
    fire(0, 0)

    def chunk_body(k, carry):
        slot = lax.rem(k, 2)

        @pl.when(k + 1 < _NCH)
        def _():
            fire(k + 1, 1 - slot)

        dummy = tab_sh.at[pl.ds(0, _CH)]
        pltpu.make_async_copy(dummy, hbuf.at[slot], sems.at[slot, 0]).wait()
        pltpu.make_async_copy(dummy, rbuf.at[slot], sems.at[slot, 1]).wait()
        pltpu.make_async_copy(dummy, tbuf.at[slot], sems.at[slot, 2]).wait()

        def sample_body(j, carry2):
            terms = []
            for v in range(_D // 32):
                e2 = plsc.bitcast(hbuf[slot, j, pl.ds(v * _L, _L)], jnp.bfloat16)
                r2 = plsc.bitcast(rbuf[slot, j, pl.ds(v * _L, _L)], jnp.bfloat16)
                t2 = plsc.bitcast(tbuf[slot, j, pl.ds(v * _L, _L)], jnp.bfloat16)
                u2 = jnp.abs(e2 + r2 - t2)
                ua, ub = plsc.unpack(u2, format=plsc.PackFormat.INTERLEAVED)
                terms.append(ua)
                terms.append(ub)
            while len(terms) > 1:
                terms = [terms[i] + terms[i + 1]
                         for i in range(0, len(terms), 2)]
            ubuf[pl.ds(j * 17, _L)] = terms[0]
            return carry2

        lax.fori_loop(0, _CH, sample_body, 0, unroll=2)

        iota17 = lax.iota(jnp.int32, _L) * 17

        def red_body(gi, carry2):
            cols = [plsc.load_gather(ubuf, [iota17 + (gi * _L * 17 + j)])
                    for j in range(_L)]
            while len(cols) > 1:
                cols = [cols[i] + cols[i + 1] for i in range(0, len(cols), 2)]
            out_v[pl.ds(k * _CH + gi * _L, _L)] = jnp.float32(_GAMMA) - cols[0]
            return carry2

        lax.fori_loop(0, _CH // _L, red_body, 0)
        return carry

    lax.fori_loop(0, _NCH, chunk_body, 0)

    pltpu.sync_copy(out_v, out_hbm.at[pl.ds(base, _PT)])


def kernel(sample, entity_embedding, relation_embedding):
    mesh = plsc.VectorSubcoreMesh(core_axis_name="c", subcore_axis_name="s")
    call = pl.kernel(
        _body,
        out_type=jax.ShapeDtypeStruct((_B,), jnp.float32),
        mesh=mesh,
        compiler_params=pltpu.CompilerParams(
            needs_layout_passes=False, use_tc_tiling_on_sc=False),
        scratch_types=[
            pltpu.VMEM((3, _PT), jnp.int32),
            pltpu.VMEM((_PT,), jnp.int32),
            pltpu.VMEM((_PT,), jnp.int32),
            pltpu.VMEM((_PT,), jnp.int32),
            pltpu.VMEM((2, _CH, _D // 2), jnp.int32),
            pltpu.VMEM((2, _CH, _D // 2), jnp.int32),
            pltpu.VMEM((2, _CH, _D // 2), jnp.int32),
            pltpu.VMEM((_CH * 17,), jnp.float32),
            pltpu.VMEM((_PT,), jnp.float32),
            pltpu.VMEM((_RT, _D), jnp.float32),
            pltpu.VMEM((_RT, _D // 2), jnp.int32),
            pltpu.VMEM_SHARED((2 * _NE, _D // 2), jnp.int32),
            pltpu.SemaphoreType.DMA((2, 3)),
        ],
    )
    score = call(sample.T, entity_embedding, relation_embedding)
    return score.reshape(_B, 1)
